# baseline (speedup 1.0000x reference)
import functools

import jax
import jax.numpy as jnp
from jax import lax
from jax.experimental import pallas as pl
from jax.experimental.pallas import tpu as pltpu
from jax.experimental.pallas import tpu_sc as plsc

_DIM = 32
_K = 8192
_TILE_N = 1024

_SC_CORES = 2
_SC_SUBCORES = 16
_ROW = 128


def _argmax_body(x_ref, emb_ref, idx_ref, tab_ref, en_ref):
    i = pl.program_id(0)

    @pl.when(i == 0)
    def _():
        emb = emb_ref[...]
        en_ref[...] = emb / jnp.maximum(
            jnp.sqrt(jnp.sum(emb * emb, axis=-1, keepdims=True)), 1e-12)

    x = x_ref[0]
    xn = x / jnp.maximum(jnp.sqrt(jnp.sum(x * x, axis=-1, keepdims=True)), 1e-12)
    dist = jax.lax.dot_general(xn, en_ref[...], (((1,), (1,)), ((), ())),
                               preferred_element_type=jnp.float32)
    idx_ref[0, 0] = jnp.argmax(dist, axis=-1).astype(jnp.int32)
    rows = emb_ref[pl.ds(i * _TILE_N, _TILE_N), :]
    tab_ref[...] = jnp.pad(rows, ((0, 0), (0, _ROW - _DIM)))


def _nearest_code_indices(x3, embeddings, nb):
    return pl.pallas_call(
        _argmax_body,
        grid=(nb,),
        in_specs=[
            pl.BlockSpec((1, _TILE_N, _DIM), lambda i: (i, 0, 0)),
            pl.BlockSpec((_K, _DIM), lambda i: (0, 0)),
        ],
        out_specs=[
            pl.BlockSpec((1, 1, _TILE_N), lambda i: (i, 0, 0)),
            pl.BlockSpec((_TILE_N, _ROW), lambda i: (i, 0)),
        ],
        out_shape=[
            jax.ShapeDtypeStruct((nb, 1, _TILE_N), jnp.int32),
            jax.ShapeDtypeStruct((_K, _ROW), jnp.float32),
        ],
        scratch_shapes=[pltpu.VMEM((_K, _DIM), jnp.float32)],
        compiler_params=pltpu.CompilerParams(allow_input_fusion=[True, True]),
    )(x3, embeddings)


def _make_sc_gather(n_rows, nb):
    nw = _SC_CORES * _SC_SUBCORES
    b_per_w = n_rows // nw
    chunk = 128
    n_chunks = b_per_w // chunk
    blocks_per_w = _TILE_N // b_per_w
    mesh = plsc.VectorSubcoreMesh(core_axis_name="c", subcore_axis_name="s")

    @functools.partial(
        pl.kernel, mesh=mesh,
        out_type=jax.ShapeDtypeStruct((n_rows, _ROW), jnp.float32),
        scratch_types=[
            pltpu.VMEM((b_per_w,), jnp.int32),
            pltpu.VMEM((b_per_w, _ROW), jnp.float32),
            pltpu.SemaphoreType.DMA,
        ],
    )
    def gather_rows(table_hbm, idx_hbm, out_hbm, idx_v, rows_v, sem):
        wid = lax.axis_index("s") * _SC_CORES + lax.axis_index("c")
        blk = wid // blocks_per_w
        off = (wid % blocks_per_w) * b_per_w
        pltpu.sync_copy(idx_hbm.at[blk, 0, pl.ds(off, b_per_w)], idx_v)
        copies = [
            pltpu.async_copy(table_hbm.at[idx_v.at[pl.ds(j * chunk, chunk)]],
                             rows_v.at[pl.ds(j * chunk, chunk)], sem)
            for j in range(n_chunks)
        ]
        for c in copies:
            c.wait()
        base = wid * b_per_w
        pltpu.sync_copy(rows_v, out_hbm.at[pl.ds(base, b_per_w)])

    return gather_rows


def kernel(x, embeddings):
    shape = x.shape
    n = x.size // shape[-1]
    nb = n // _TILE_N
    x3 = x.reshape(nb, _TILE_N, _DIM)
    idx, table = _nearest_code_indices(x3, embeddings, nb)
    quantized = _make_sc_gather(n, nb)(table, idx)[:, :_DIM]
    return quantized.reshape(shape), idx.reshape(shape[:-1])


# speedup vs baseline: 1.0014x; 1.0014x over previous
import functools

import jax
import jax.numpy as jnp
from jax import lax
from jax.experimental import pallas as pl
from jax.experimental.pallas import tpu as pltpu
from jax.experimental.pallas import tpu_sc as plsc

_DIM = 32
_K = 8192
_TILE_N = 1024

_SC_CORES = 2
_SC_SUBCORES = 16
_ROW = 128


def _argmax_body(x_ref, emb_ref, idx_ref, tab_ref, en_ref):
    i = pl.program_id(0)

    @pl.when(i == 0)
    def _():
        emb = emb_ref[...]
        en_ref[...] = emb / jnp.maximum(
            jnp.sqrt(jnp.sum(emb * emb, axis=-1, keepdims=True)), 1e-12)

    x = x_ref[0]
    xn = x / jnp.maximum(jnp.sqrt(jnp.sum(x * x, axis=-1, keepdims=True)), 1e-12)
    dist = jax.lax.dot_general(xn, en_ref[...], (((1,), (1,)), ((), ())),
                               preferred_element_type=jnp.float32)
    idx_ref[0, 0] = jnp.argmax(dist, axis=-1).astype(jnp.int32)
    rows = emb_ref[pl.ds(i * _TILE_N, _TILE_N), :]
    tab_ref[...] = jnp.pad(rows, ((0, 0), (0, _ROW - _DIM)))


def _nearest_code_indices(x3, embeddings, nb):
    return pl.pallas_call(
        _argmax_body,
        grid=(nb,),
        in_specs=[
            pl.BlockSpec((1, _TILE_N, _DIM), lambda i: (i, 0, 0)),
            pl.BlockSpec((_K, _DIM), lambda i: (0, 0)),
        ],
        out_specs=[
            pl.BlockSpec((1, 1, _TILE_N), lambda i: (i, 0, 0)),
            pl.BlockSpec((_TILE_N, _ROW), lambda i: (i, 0)),
        ],
        out_shape=[
            jax.ShapeDtypeStruct((nb, 1, _TILE_N), jnp.int32),
            jax.ShapeDtypeStruct((_K, _ROW), jnp.float32),
        ],
        scratch_shapes=[pltpu.VMEM((_K, _DIM), jnp.float32)],
    )(x3, embeddings)


def _make_sc_gather(n_rows, nb):
    nw = _SC_CORES * _SC_SUBCORES
    b_per_w = n_rows // nw
    chunk = 128
    n_chunks = b_per_w // chunk
    blocks_per_w = _TILE_N // b_per_w
    mesh = plsc.VectorSubcoreMesh(core_axis_name="c", subcore_axis_name="s")

    tok_cols = 1024

    @functools.partial(
        pl.kernel, mesh=mesh,
        out_type=[
            jax.ShapeDtypeStruct((n_rows, _ROW), jnp.float32),
            jax.ShapeDtypeStruct((n_rows // tok_cols, tok_cols), jnp.int32),
        ],
        scratch_types=[
            pltpu.VMEM((b_per_w,), jnp.int32),
            pltpu.VMEM((b_per_w, _ROW), jnp.float32),
            pltpu.SemaphoreType.DMA,
        ],
    )
    def gather_rows(table_hbm, idx_hbm, out_hbm, oidx_hbm, idx_v, rows_v, sem):
        wid = lax.axis_index("s") * _SC_CORES + lax.axis_index("c")
        blk = wid // blocks_per_w
        off = (wid % blocks_per_w) * b_per_w
        pltpu.sync_copy(idx_hbm.at[blk, 0, pl.ds(off, b_per_w)], idx_v)
        copies = [
            pltpu.async_copy(table_hbm.at[idx_v.at[pl.ds(j * chunk, chunk)]],
                             rows_v.at[pl.ds(j * chunk, chunk)], sem)
            for j in range(n_chunks)
        ]
        base = wid * b_per_w
        row = base // tok_cols
        col = base % tok_cols
        pltpu.sync_copy(idx_v, oidx_hbm.at[row, pl.ds(col, b_per_w)])
        for c in copies:
            c.wait()
        pltpu.sync_copy(rows_v, out_hbm.at[pl.ds(base, b_per_w)])

    return gather_rows


def kernel(x, embeddings):
    shape = x.shape
    n = x.size // shape[-1]
    nb = n // _TILE_N
    x3 = x.reshape(nb, _TILE_N, _DIM)
    idx, table = _nearest_code_indices(x3, embeddings, nb)
    quantized, oidx = _make_sc_gather(n, nb)(table, idx)
    return quantized[:, :_DIM].reshape(shape), oidx.reshape(shape[:-1])
